# Initial kernel scaffold; baseline (speedup 1.0000x reference)
import jax
import jax.numpy as jnp
from jax.experimental import pallas as pl


def kernel(features, W1, gamma, beta, W2, b2):
    raise NotImplementedError("write your pallas kernel here")


# speedup vs baseline: 1.4775x; 1.4775x over previous
import jax
import jax.numpy as jnp
from jax.experimental import pallas as pl
from jax.experimental.pallas import tpu as pltpu

A_ = 5
C_ = 80
CIN = 768
CH = 1024
COUT = A_ * (5 + C_)
EPS = 1e-5
H = 32
W = 32
NPIX = H * W


def _conv1_body(x_ref, w_ref, y_ref, ps_ref, pq_ref, acc_ref):
    x = x_ref[0]
    first = True
    for dx in range(3):
        xd = x[:, dx:dx + W, :].reshape((H + 2) * W, CIN)
        for dy in range(3):
            xm = xd[dy * W:dy * W + NPIX, :]
            d = jnp.dot(xm, w_ref[dy, dx],
                        preferred_element_type=jnp.float32)
            if first:
                acc_ref[...] = d
                first = False
            else:
                acc_ref[...] += d
    acc = acc_ref[...]
    ps_ref[0, 0] = jnp.sum(acc, axis=0)
    pq_ref[0, 0] = jnp.sum(acc * acc, axis=0)
    y_ref[0] = acc.astype(jnp.bfloat16).reshape(H, W, CH)


def _conv2_body(y_ref, sc_ref, sh_ref, w2_ref, b2_ref, o_ref):
    y = y_ref[0].reshape(NPIX, CH).astype(jnp.float32)
    z = y * sc_ref[0] + sh_ref[0]
    z = jnp.where(z > 0, z, 0.1 * z).astype(jnp.bfloat16)
    o = jnp.dot(z, w2_ref[...], preferred_element_type=jnp.float32)
    o_ref[0] = (o + b2_ref[0]).reshape(H, W, COUT)


def kernel(features, W1, gamma, beta, W2, b2):
    B = features.shape[0]
    x = jnp.transpose(features, (0, 2, 3, 1))
    x = jnp.pad(x, ((0, 0), (1, 1), (1, 1), (0, 0))).astype(jnp.bfloat16)
    w1 = jnp.transpose(W1, (2, 3, 1, 0)).astype(jnp.bfloat16)
    w2 = jnp.transpose(W2[:, :, 0, 0]).astype(jnp.bfloat16)

    y, ps, pq = pl.pallas_call(
        _conv1_body,
        grid=(B,),
        in_specs=[
            pl.BlockSpec((1, H + 2, W + 2, CIN), lambda b: (b, 0, 0, 0)),
            pl.BlockSpec((3, 3, CIN, CH), lambda b: (0, 0, 0, 0)),
        ],
        out_specs=[
            pl.BlockSpec((1, H, W, CH), lambda b: (b, 0, 0, 0)),
            pl.BlockSpec((1, 1, CH), lambda b: (b, 0, 0)),
            pl.BlockSpec((1, 1, CH), lambda b: (b, 0, 0)),
        ],
        out_shape=[
            jax.ShapeDtypeStruct((B, H, W, CH), jnp.bfloat16),
            jax.ShapeDtypeStruct((B, 1, CH), jnp.float32),
            jax.ShapeDtypeStruct((B, 1, CH), jnp.float32),
        ],
        scratch_shapes=[pltpu.VMEM((NPIX, CH), jnp.float32)],
        compiler_params=pltpu.CompilerParams(
            dimension_semantics=("arbitrary",)),
    )(x, w1)

    n = jnp.float32(B * NPIX)
    mean = jnp.sum(ps[:, 0, :], axis=0) / n
    var = jnp.sum(pq[:, 0, :], axis=0) / n - mean * mean
    scale = gamma * jax.lax.rsqrt(var + EPS)
    shift = beta - mean * scale

    out = pl.pallas_call(
        _conv2_body,
        grid=(B,),
        in_specs=[
            pl.BlockSpec((1, H, W, CH), lambda b: (b, 0, 0, 0)),
            pl.BlockSpec((1, CH), lambda b: (0, 0)),
            pl.BlockSpec((1, CH), lambda b: (0, 0)),
            pl.BlockSpec((CH, COUT), lambda b: (0, 0)),
            pl.BlockSpec((1, COUT), lambda b: (0, 0)),
        ],
        out_specs=pl.BlockSpec((1, H, W, COUT), lambda b: (b, 0, 0, 0)),
        out_shape=jax.ShapeDtypeStruct((B, H, W, COUT), jnp.float32),
        compiler_params=pltpu.CompilerParams(
            dimension_semantics=("arbitrary",)),
    )(y, scale.reshape(1, CH), shift.reshape(1, CH), w2,
      b2.reshape(1, COUT))

    return out
